# baseline (speedup 1.0000x reference)
import functools

import jax
import jax.numpy as jnp
from jax import lax
from jax.experimental import pallas as pl
from jax.experimental.pallas import tpu as pltpu
from jax.experimental.pallas import tpu_sc as plsc

N_SH = 8
FEAT = 4096
BATCH = 1024
NC, NS, L = 2, 16, 16
NW = NC * NS
JPW = FEAT // NW
KJ = 2
NCHUNK = JPW // KJ
CVR = BATCH // L
NBUF = 4


def _sc_shuffle(xT, idx_flat, wsp, *, interpret=False):
    mesh = plsc.VectorSubcoreMesh(
        core_axis_name="c", subcore_axis_name="s",
        num_cores=NC, num_subcores=NS)

    GR = KJ * N_SH

    @functools.partial(
        pl.kernel,
        out_type=jax.ShapeDtypeStruct((FEAT, BATCH), jnp.float32),
        mesh=mesh,
        scratch_types=[
            pltpu.VMEM((JPW * N_SH,), jnp.int32),
            pltpu.VMEM((JPW * (N_SH + 1) * L,), jnp.float32),
            pltpu.VMEM((NBUF, GR, BATCH), jnp.float32),
            pltpu.VMEM((2, KJ, BATCH), jnp.float32),
            [pltpu.SemaphoreType.DMA] * NBUF,
            [pltpu.SemaphoreType.DMA] * 2,
        ],
        interpret=interpret,
    )
    def body(xT_hbm, idx_hbm, wsp_hbm, out_hbm,
             idx_v, wsp_v, rows_v, stage_v, gsem, ssem):
        wid = lax.axis_index("s") * NC + lax.axis_index("c")
        jbase = wid * JPW
        pltpu.sync_copy(idx_hbm.at[pl.ds(jbase * N_SH, JPW * N_SH)], idx_v)
        pltpu.sync_copy(wsp_hbm.at[pl.ds(jbase * (N_SH + 1) * L,
                                         JPW * (N_SH + 1) * L)], wsp_v)

        def start_gather(c, b):
            pltpu.async_copy(
                xT_hbm.at[idx_v.at[pl.ds(c * GR, GR)]], rows_v.at[b], gsem[b])

        start_gather(0, 0)
        start_gather(1, 1)

        @pl.loop(0, NCHUNK, step=NBUF)
        def _chunk(c):
            for b in range(NBUF):
                cc = c + b
                @pl.when(cc + 2 < NCHUNK)
                def _():
                    start_gather(cc + 2, (b + 2) % NBUF)
                pltpu.make_async_copy(
                    xT_hbm.at[pl.ds(0, GR)], rows_v.at[b], gsem[b]).wait()
                sb = b % 2
                @pl.when(cc >= 2)
                def _():
                    pltpu.make_async_copy(
                        stage_v.at[sb], out_hbm.at[pl.ds(jbase, KJ)],
                        ssem[sb]).wait()
                for jj in range(KJ):
                    jloc = cc * KJ + jj
                    wv = [wsp_v[pl.ds((jloc * (N_SH + 1) + i) * L, L)] * 0.125
                          for i in range(N_SH)]
                    bv = wsp_v[pl.ds((jloc * (N_SH + 1) + N_SH) * L, L)]
                    for ch in range(CVR):
                        acc = bv + rows_v[b, jj * N_SH, pl.ds(ch * L, L)] * wv[0]
                        for i in range(1, N_SH):
                            acc = acc + rows_v[b, jj * N_SH + i,
                                               pl.ds(ch * L, L)] * wv[i]
                        stage_v[sb, jj, pl.ds(ch * L, L)] = acc
                pltpu.async_copy(
                    stage_v.at[sb], out_hbm.at[pl.ds(jbase + cc * KJ, KJ)],
                    ssem[sb])

        for sb in range(2):
            pltpu.make_async_copy(
                stage_v.at[sb], out_hbm.at[pl.ds(jbase, KJ)], ssem[sb]).wait()

    return body(xT, idx_flat, wsp)


def kernel(x, weights, bias, perms):
    xT = x.T
    idx_flat = perms.T.reshape(-1)
    wsp = jnp.broadcast_to(
        jnp.concatenate([weights.T, bias[:, None]], axis=1).reshape(-1)[:, None],
        (FEAT * (N_SH + 1), L)).reshape(-1)
    outT = _sc_shuffle(xT, idx_flat, wsp)
    return outT.T


# speedup vs baseline: 1.1395x; 1.1395x over previous
import functools

import jax
import jax.numpy as jnp
from jax import lax
from jax.experimental import pallas as pl
from jax.experimental.pallas import tpu as pltpu
from jax.experimental.pallas import tpu_sc as plsc

N_SH = 8
FEAT = 4096
BATCH = 1024
NC, NS, L = 2, 16, 16
NW = NC * NS
JPW = FEAT // NW
KJ = 2
NCHUNK = JPW // KJ
CVR = BATCH // L
NBUF = 4


def _sc_shuffle(xT, idx_flat, wsp, *, interpret=False):
    mesh = plsc.VectorSubcoreMesh(
        core_axis_name="c", subcore_axis_name="s",
        num_cores=NC, num_subcores=NS)

    GR = KJ * N_SH

    @functools.partial(
        pl.kernel,
        out_type=jax.ShapeDtypeStruct((FEAT, BATCH), jnp.float32),
        mesh=mesh,
        scratch_types=[
            pltpu.VMEM((JPW * N_SH,), jnp.int32),
            pltpu.VMEM((JPW * L,), jnp.float32),
            pltpu.VMEM((NBUF, GR, BATCH), jnp.float32),
            pltpu.VMEM((2, KJ, BATCH), jnp.float32),
            [pltpu.SemaphoreType.DMA] * NBUF,
            [pltpu.SemaphoreType.DMA] * 2,
        ],
        interpret=interpret,
    )
    def body(xT_hbm, idx_hbm, wsp_hbm, out_hbm,
             idx_v, wsp_v, rows_v, stage_v, gsem, ssem):
        wid = lax.axis_index("s") * NC + lax.axis_index("c")
        jbase = wid * JPW
        pltpu.sync_copy(idx_hbm.at[pl.ds(jbase * N_SH, JPW * N_SH)], idx_v)
        pltpu.sync_copy(wsp_hbm.at[pl.ds(jbase * L, JPW * L)], wsp_v)

        def start_gather(c, b):
            pltpu.async_copy(
                xT_hbm.at[idx_v.at[pl.ds(c * GR, GR)]], rows_v.at[b], gsem[b])

        start_gather(0, 0)
        start_gather(1, 1)

        @pl.loop(0, NCHUNK, step=NBUF)
        def _chunk(c):
            for b in range(NBUF):
                cc = c + b
                @pl.when(cc + 2 < NCHUNK)
                def _():
                    start_gather(cc + 2, (b + 2) % NBUF)
                pltpu.make_async_copy(
                    xT_hbm.at[pl.ds(0, GR)], rows_v.at[b], gsem[b]).wait()
                sb = b % 2
                @pl.when(cc >= 2)
                def _():
                    pltpu.make_async_copy(
                        stage_v.at[sb], out_hbm.at[pl.ds(jbase, KJ)],
                        ssem[sb]).wait()
                for jj in range(KJ):
                    jloc = cc * KJ + jj
                    wbv = wsp_v[pl.ds(jloc * L, L)]
                    ws = [wbv[i] * 0.125 for i in range(N_SH)]
                    bsc = wbv[N_SH]
                    for ch in range(CVR):
                        acc = jnp.full((L,), bsc, jnp.float32)
                        for i in range(N_SH):
                            acc = acc + rows_v[b, jj * N_SH + i,
                                               pl.ds(ch * L, L)] * ws[i]
                        stage_v[sb, jj, pl.ds(ch * L, L)] = acc
                pltpu.async_copy(
                    stage_v.at[sb], out_hbm.at[pl.ds(jbase + cc * KJ, KJ)],
                    ssem[sb])

        for sb in range(2):
            pltpu.make_async_copy(
                stage_v.at[sb], out_hbm.at[pl.ds(jbase, KJ)], ssem[sb]).wait()

    return body(xT, idx_flat, wsp)


def kernel(x, weights, bias, perms):
    xT = x.T
    idx_flat = perms.T.reshape(-1)
    wsp = jnp.concatenate(
        [weights.T, bias[:, None],
         jnp.zeros((FEAT, L - N_SH - 1), jnp.float32)], axis=1).reshape(-1)
    outT = _sc_shuffle(xT, idx_flat, wsp)
    return outT.T


# speedup vs baseline: 1.2907x; 1.1326x over previous
import functools

import jax
import jax.numpy as jnp
from jax import lax
from jax.experimental import pallas as pl
from jax.experimental.pallas import tpu as pltpu
from jax.experimental.pallas import tpu_sc as plsc

N_SH = 8
FEAT = 4096
BATCH = 1024
NC, NS, L = 2, 16, 16
NW = NC * NS
JPW = FEAT // NW
KJ = 2
NCHUNK = JPW // KJ
CVR = BATCH // L
NBUF = 2


def _sc_shuffle(xT, idx_flat, wsp, *, interpret=False):
    mesh = plsc.VectorSubcoreMesh(
        core_axis_name="c", subcore_axis_name="s",
        num_cores=NC, num_subcores=NS)

    GR = KJ * N_SH

    @functools.partial(
        pl.kernel,
        out_type=jax.ShapeDtypeStruct((FEAT, BATCH), jnp.float32),
        mesh=mesh,
        scratch_types=[
            pltpu.VMEM((JPW * N_SH,), jnp.int32),
            pltpu.VMEM((JPW * L,), jnp.float32),
            pltpu.VMEM((NBUF, GR, BATCH), jnp.float32),
            pltpu.VMEM((2, KJ, BATCH), jnp.float32),
            [pltpu.SemaphoreType.DMA] * NBUF,
            [pltpu.SemaphoreType.DMA] * 2,
        ],
        interpret=interpret,
    )
    def body(xT_hbm, idx_hbm, wsp_hbm, out_hbm,
             idx_v, wsp_v, rows_v, stage_v, gsem, ssem):
        wid = lax.axis_index("s") * NC + lax.axis_index("c")
        jbase = wid * JPW
        pltpu.sync_copy(idx_hbm.at[pl.ds(jbase * N_SH, JPW * N_SH)], idx_v)
        pltpu.sync_copy(wsp_hbm.at[pl.ds(jbase * L, JPW * L)], wsp_v)

        def start_gather(c, b):
            pltpu.async_copy(
                xT_hbm.at[idx_v.at[pl.ds(c * GR, GR)]], rows_v.at[b], gsem[b])

        start_gather(0, 0)

        @pl.loop(0, NCHUNK, step=NBUF)
        def _chunk(c):
            for b in range(NBUF):
                cc = c + b
                @pl.when(cc + 1 < NCHUNK)
                def _():
                    start_gather(cc + 1, (b + 1) % NBUF)
                pltpu.make_async_copy(
                    xT_hbm.at[pl.ds(0, GR)], rows_v.at[b], gsem[b]).wait()
                sb = b % 2
                @pl.when(cc >= 2)
                def _():
                    pltpu.make_async_copy(
                        stage_v.at[sb], out_hbm.at[pl.ds(jbase, KJ)],
                        ssem[sb]).wait()
                for jj in range(KJ):
                    jloc = cc * KJ + jj
                    wbv = wsp_v[pl.ds(jloc * L, L)]
                    ws = [wbv[i] * 0.125 for i in range(N_SH)]
                    bsc = wbv[N_SH]
                    for ch in range(CVR):
                        acc = jnp.full((L,), bsc, jnp.float32)
                        for i in range(N_SH):
                            acc = acc + rows_v[b, jj * N_SH + i,
                                               pl.ds(ch * L, L)] * ws[i]
                        stage_v[sb, jj, pl.ds(ch * L, L)] = acc
                pltpu.async_copy(
                    stage_v.at[sb], out_hbm.at[pl.ds(jbase + cc * KJ, KJ)],
                    ssem[sb])

        for sb in range(2):
            pltpu.make_async_copy(
                stage_v.at[sb], out_hbm.at[pl.ds(jbase, KJ)], ssem[sb]).wait()

    return body(xT, idx_flat, wsp)


def kernel(x, weights, bias, perms):
    xT = x.T
    idx_flat = perms.T.reshape(-1)
    wsp = jnp.concatenate(
        [weights.T, bias[:, None],
         jnp.zeros((FEAT, L - N_SH - 1), jnp.float32)], axis=1).reshape(-1)
    outT = _sc_shuffle(xT, idx_flat, wsp)
    return outT.T


# speedup vs baseline: 2.2813x; 1.7676x over previous
import functools

import jax
import jax.numpy as jnp
from jax import lax
from jax.experimental import pallas as pl
from jax.experimental.pallas import tpu as pltpu
from jax.experimental.pallas import tpu_sc as plsc

N_SH = 8
FEAT = 4096
BATCH = 1024
NC, NS, L = 2, 16, 16
NW = NC * NS
JPW = FEAT // NW
KJ = 2
NCHUNK = JPW // KJ
CVR = BATCH // L
NBUF = 2


def _sc_shuffle(xT, idx_flat, wsp, *, interpret=False):
    mesh = plsc.VectorSubcoreMesh(
        core_axis_name="c", subcore_axis_name="s",
        num_cores=NC, num_subcores=NS)

    GR = KJ * N_SH

    @functools.partial(
        pl.kernel,
        out_type=jax.ShapeDtypeStruct((FEAT, BATCH), jnp.float32),
        mesh=mesh,
        scratch_types=[
            pltpu.VMEM((JPW * N_SH,), jnp.int32),
            pltpu.VMEM((JPW * L,), jnp.float32),
            pltpu.VMEM((NBUF, GR, BATCH), jnp.float32),
            pltpu.VMEM((2, KJ, BATCH), jnp.float32),
            [pltpu.SemaphoreType.DMA] * NBUF,
            [pltpu.SemaphoreType.DMA] * 2,
        ],
        interpret=interpret,
    )
    def body(xT_hbm, idx_hbm, wsp_hbm, out_hbm,
             idx_v, wsp_v, rows_v, stage_v, gsem, ssem):
        wid = lax.axis_index("s") * NC + lax.axis_index("c")
        jbase = wid * JPW
        pltpu.sync_copy(idx_hbm.at[pl.ds(jbase * N_SH, JPW * N_SH)], idx_v)
        pltpu.sync_copy(wsp_hbm.at[pl.ds(jbase * L, JPW * L)], wsp_v)

        def start_gather(c, b):
            pltpu.async_copy(
                xT_hbm.at[idx_v.at[pl.ds(c * GR, GR)]], rows_v.at[b], gsem[b])

        start_gather(0, 0)

        @pl.loop(0, NCHUNK, step=NBUF)
        def _chunk(c):
            for b in range(NBUF):
                cc = c + b
                @pl.when(cc + 1 < NCHUNK)
                def _():
                    start_gather(cc + 1, (b + 1) % NBUF)
                pltpu.make_async_copy(
                    xT_hbm.at[pl.ds(0, GR)], rows_v.at[b], gsem[b]).wait()
                sb = b % 2
                @pl.when(cc >= 2)
                def _():
                    pltpu.make_async_copy(
                        stage_v.at[sb], out_hbm.at[pl.ds(jbase, KJ)],
                        ssem[sb]).wait()
                for jj in range(KJ):
                    jloc = cc * KJ + jj
                    wbv = wsp_v[pl.ds(jloc * L, L)]
                    ws = [wbv[i] * 0.125 for i in range(N_SH)]
                    bsc = wbv[N_SH]

                    @plsc.parallel_loop(0, CVR, unroll=8)
                    def _ch(ch):
                        acc = jnp.full((L,), bsc, jnp.float32)
                        for i in range(N_SH):
                            acc = acc + rows_v[b, jj * N_SH + i,
                                               pl.ds(ch * L, L)] * ws[i]
                        stage_v[sb, jj, pl.ds(ch * L, L)] = acc
                pltpu.async_copy(
                    stage_v.at[sb], out_hbm.at[pl.ds(jbase + cc * KJ, KJ)],
                    ssem[sb])

        for sb in range(2):
            pltpu.make_async_copy(
                stage_v.at[sb], out_hbm.at[pl.ds(jbase, KJ)], ssem[sb]).wait()

    return body(xT, idx_flat, wsp)


def kernel(x, weights, bias, perms):
    xT = x.T
    idx_flat = perms.T.reshape(-1)
    wsp = jnp.concatenate(
        [weights.T, bias[:, None],
         jnp.zeros((FEAT, L - N_SH - 1), jnp.float32)], axis=1).reshape(-1)
    outT = _sc_shuffle(xT, idx_flat, wsp)
    return outT.T


# speedup vs baseline: 2.5726x; 1.1277x over previous
import functools

import jax
import jax.numpy as jnp
from jax import lax
from jax.experimental import pallas as pl
from jax.experimental.pallas import tpu as pltpu
from jax.experimental.pallas import tpu_sc as plsc

N_SH = 8
FEAT = 4096
BATCH = 1024
NC, NS, L = 2, 16, 16
NW = NC * NS
JPW = FEAT // NW
KJ = 2
NCHUNK = JPW // KJ
CVR = BATCH // L
NBUF = 4


def _sc_shuffle(xT, idx_flat, wsp, *, interpret=False):
    mesh = plsc.VectorSubcoreMesh(
        core_axis_name="c", subcore_axis_name="s",
        num_cores=NC, num_subcores=NS)

    GR = KJ * N_SH

    @functools.partial(
        pl.kernel,
        out_type=jax.ShapeDtypeStruct((FEAT, BATCH), jnp.float32),
        mesh=mesh,
        scratch_types=[
            pltpu.VMEM((JPW * N_SH,), jnp.int32),
            pltpu.VMEM((JPW * L,), jnp.float32),
            pltpu.VMEM((NBUF, GR, BATCH), jnp.float32),
            pltpu.VMEM((2, KJ, BATCH), jnp.float32),
            [pltpu.SemaphoreType.DMA] * NBUF,
            [pltpu.SemaphoreType.DMA] * 2,
        ],
        interpret=interpret,
    )
    def body(xT_hbm, idx_hbm, wsp_hbm, out_hbm,
             idx_v, wsp_v, rows_v, stage_v, gsem, ssem):
        wid = lax.axis_index("s") * NC + lax.axis_index("c")
        jbase = wid * JPW
        pltpu.sync_copy(idx_hbm.at[pl.ds(jbase * N_SH, JPW * N_SH)], idx_v)
        pltpu.sync_copy(wsp_hbm.at[pl.ds(jbase * L, JPW * L)], wsp_v)

        def start_gather(c, b):
            pltpu.async_copy(
                xT_hbm.at[idx_v.at[pl.ds(c * GR, GR)]], rows_v.at[b], gsem[b])

        start_gather(0, 0)
        start_gather(1, 1)

        @pl.loop(0, NCHUNK, step=NBUF)
        def _chunk(c):
            for b in range(NBUF):
                cc = c + b
                @pl.when(cc + 2 < NCHUNK)
                def _():
                    start_gather(cc + 2, (b + 2) % NBUF)
                pltpu.make_async_copy(
                    xT_hbm.at[pl.ds(0, GR)], rows_v.at[b], gsem[b]).wait()
                sb = b % 2
                @pl.when(cc >= 2)
                def _():
                    pltpu.make_async_copy(
                        stage_v.at[sb], out_hbm.at[pl.ds(jbase, KJ)],
                        ssem[sb]).wait()
                for jj in range(KJ):
                    jloc = cc * KJ + jj
                    wbv = wsp_v[pl.ds(jloc * L, L)]
                    ws = [wbv[i] * 0.125 for i in range(N_SH)]
                    bsc = wbv[N_SH]

                    @plsc.parallel_loop(0, CVR, unroll=8)
                    def _ch(ch):
                        acc = jnp.full((L,), bsc, jnp.float32)
                        for i in range(N_SH):
                            acc = acc + rows_v[b, jj * N_SH + i,
                                               pl.ds(ch * L, L)] * ws[i]
                        stage_v[sb, jj, pl.ds(ch * L, L)] = acc
                pltpu.async_copy(
                    stage_v.at[sb], out_hbm.at[pl.ds(jbase + cc * KJ, KJ)],
                    ssem[sb])

        for sb in range(2):
            pltpu.make_async_copy(
                stage_v.at[sb], out_hbm.at[pl.ds(jbase, KJ)], ssem[sb]).wait()

    return body(xT, idx_flat, wsp)


def kernel(x, weights, bias, perms):
    xT = x.T
    idx_flat = perms.T.reshape(-1)
    wsp = jnp.concatenate(
        [weights.T, bias[:, None],
         jnp.zeros((FEAT, L - N_SH - 1), jnp.float32)], axis=1).reshape(-1)
    outT = _sc_shuffle(xT, idx_flat, wsp)
    return outT.T
